# baseline (speedup 1.0000x reference)
import functools

import jax
import jax.numpy as jnp
from jax import lax
from jax.experimental import pallas as pl
from jax.experimental.pallas import tpu as pltpu
from jax.experimental.pallas import tpu_sc as plsc

_HIDDEN = 2048
_E = 64
_T = 16384
_BT = 1024
_NCHUNK = 4
_CT = _T // _NCHUNK
_NW = 32
_TPW = _CT // _NW
_L = 16
_NG = _TPW // _L
_NC = 4


def _tc_logits_body(x_ref, w_ref, o_ref, p_ref):
    lt = lax.dot_general(
        w_ref[...].astype(jnp.bfloat16), x_ref[...].astype(jnp.bfloat16),
        dimension_numbers=(((1,), (1,)), ((), ())),
        preferred_element_type=jnp.float32,
        precision=lax.Precision.DEFAULT,
    )
    l = lt.T
    o_ref[...] = l
    m = jnp.max(l, axis=1, keepdims=True)
    u = jnp.exp(l - m)
    p_ref[...] = u / jnp.sum(u, axis=1, keepdims=True)


def _tc_logits(x, W, c):
    off = c * (_CT // _BT)
    return pl.pallas_call(
        _tc_logits_body,
        grid=(_CT // _BT,),
        in_specs=[
            pl.BlockSpec((_BT, _HIDDEN), lambda i: (i + off, 0)),
            pl.BlockSpec((_E, _HIDDEN), lambda i: (0, 0)),
        ],
        out_specs=[
            pl.BlockSpec((_BT, _E), lambda i: (i, 0)),
            pl.BlockSpec((_BT, _E), lambda i: (i, 0)),
        ],
        out_shape=[
            jax.ShapeDtypeStruct((_CT, _E), jnp.float32),
            jax.ShapeDtypeStruct((_CT, _E), jnp.float32),
        ],
    )(x, W)


def _pick(ma, ia, mb, ib):
    take_a = (ma > mb) | ((ma == mb) & (ia < ib))
    return jnp.where(take_a, ma, mb), jnp.where(take_a, ia, ib)


def _merge_top2(a, b):
    m1a, i1a, m2a, i2a = a
    m1b, i1b, m2b, i2b = b
    m1, i1 = _pick(m1a, i1a, m1b, i1b)
    a_won = (m1a > m1b) | ((m1a == m1b) & (i1a < i1b))
    c1m = jnp.where(a_won, m2a, m1a)
    c1i = jnp.where(a_won, i2a, i1a)
    c2m = jnp.where(a_won, m1b, m2b)
    c2i = jnp.where(a_won, i1b, i2b)
    m2, i2 = _pick(c1m, c1i, c2m, c2i)
    return m1, i1, m2, i2


def _sc_route(probs2d):
    mesh = plsc.VectorSubcoreMesh(core_axis_name="c", subcore_axis_name="s")

    @functools.partial(
        pl.kernel,
        mesh=mesh,
        compiler_params=pltpu.CompilerParams(needs_layout_passes=False),
        out_type=[
            jax.ShapeDtypeStruct((_CT,), jnp.float32),
            jax.ShapeDtypeStruct((_CT,), jnp.float32),
            jax.ShapeDtypeStruct((_CT,), jnp.int32),
            jax.ShapeDtypeStruct((_CT,), jnp.int32),
        ],
        scratch_types=[
            pltpu.VMEM((_TPW, _E), jnp.float32),
            pltpu.VMEM((_TPW,), jnp.float32),
            pltpu.VMEM((_TPW,), jnp.float32),
            pltpu.VMEM((_TPW,), jnp.int32),
            pltpu.VMEM((_TPW,), jnp.int32),
        ],
    )
    def k(probs_hbm, w1_hbm, w2_hbm, e1_hbm, e2_hbm,
          lv, w1v, w2v, e1v, e2v):
        wid = lax.axis_index("s") * 2 + lax.axis_index("c")
        base = wid * _TPW
        pltpu.sync_copy(probs_hbm.at[pl.ds(base, _TPW)], lv)

        iota = lax.iota(jnp.int32, _L)
        neg = jnp.full((_L,), -1.0, jnp.float32)

        def group_body(g, _):
            rows = g * _L + iota
            chains = []
            for c in range(_NC):
                e0 = c * (_E // _NC)
                m1 = plsc.load_gather(lv, [rows, jnp.full((_L,), e0, jnp.int32)])
                i1 = jnp.full((_L,), e0, jnp.int32)
                m2 = neg
                i2 = i1
                for e in range(e0 + 1, e0 + _E // _NC):
                    v = plsc.load_gather(lv, [rows, jnp.full((_L,), e, jnp.int32)])
                    es = jnp.full((_L,), e, jnp.int32)
                    gt1 = v > m1
                    gt2 = v > m2
                    i2 = jnp.where(gt1, i1, jnp.where(gt2, es, i2))
                    m2 = jnp.where(gt1, m1, jnp.where(gt2, v, m2))
                    i1 = jnp.where(gt1, es, i1)
                    m1 = jnp.where(gt1, v, m1)
                chains.append((m1, i1, m2, i2))
            t01 = _merge_top2(chains[0], chains[1])
            t23 = _merge_top2(chains[2], chains[3])
            m1, i1, m2, i2 = _merge_top2(t01, t23)

            denom = (m1 + m2) + 1e-6
            w1 = m1 / denom
            w2 = m2 / denom

            w1v[pl.ds(g * _L, _L)] = w1
            w2v[pl.ds(g * _L, _L)] = w2
            e1v[pl.ds(g * _L, _L)] = i1
            e2v[pl.ds(g * _L, _L)] = i2
            return 0

        lax.fori_loop(0, _NG, group_body, 0)

        pltpu.sync_copy(w1v, w1_hbm.at[pl.ds(base, _TPW)])
        pltpu.sync_copy(w2v, w2_hbm.at[pl.ds(base, _TPW)])
        pltpu.sync_copy(e1v, e1_hbm.at[pl.ds(base, _TPW)])
        pltpu.sync_copy(e2v, e2_hbm.at[pl.ds(base, _TPW)])

    return k(probs2d)


def kernel(hidden_states, W):
    b, s, h = hidden_states.shape
    x = hidden_states.reshape(b * s, h)
    logits_c, outs_c = [], []
    for c in range(_NCHUNK):
        logits, probs = _tc_logits(x, W, c)
        logits_c.append(logits)
        outs_c.append(_sc_route(probs))
    logits = jnp.concatenate(logits_c, axis=0)
    w1, w2, e1, e2 = (jnp.concatenate([o[i] for o in outs_c]) for i in range(4))
    routing_weights = jnp.stack([w1, w2], axis=-1).reshape(b, s, 2)
    selected_experts = jnp.stack([e1, e2], axis=-1).reshape(b, s, 2)
    router_logits = logits.reshape(b, s, _E)
    return routing_weights, selected_experts, router_logits


# speedup vs baseline: 1.1067x; 1.1067x over previous
import functools

import jax
import jax.numpy as jnp
from jax import lax
from jax.experimental import pallas as pl
from jax.experimental.pallas import tpu as pltpu
from jax.experimental.pallas import tpu_sc as plsc

_HIDDEN = 2048
_E = 64
_T = 16384
_BT = 1024
_NCHUNK = 2
_CT = _T // _NCHUNK
_NW = 32
_TPW = _CT // _NW
_L = 16
_NG = _TPW // _L
_NC = 4


def _tc_logits_body(x_ref, w_ref, o_ref, p_ref):
    lt = lax.dot_general(
        w_ref[...].astype(jnp.bfloat16), x_ref[...].astype(jnp.bfloat16),
        dimension_numbers=(((1,), (1,)), ((), ())),
        preferred_element_type=jnp.float32,
        precision=lax.Precision.DEFAULT,
    )
    l = lt.T
    o_ref[...] = l
    m = jnp.max(l, axis=1, keepdims=True)
    u = jnp.exp(l - m)
    p_ref[...] = u / jnp.sum(u, axis=1, keepdims=True)


def _tc_logits(x, W, c):
    off = c * (_CT // _BT)
    return pl.pallas_call(
        _tc_logits_body,
        grid=(_CT // _BT,),
        in_specs=[
            pl.BlockSpec((_BT, _HIDDEN), lambda i: (i + off, 0)),
            pl.BlockSpec((_E, _HIDDEN), lambda i: (0, 0)),
        ],
        out_specs=[
            pl.BlockSpec((_BT, _E), lambda i: (i, 0)),
            pl.BlockSpec((_BT, _E), lambda i: (i, 0)),
        ],
        out_shape=[
            jax.ShapeDtypeStruct((_CT, _E), jnp.float32),
            jax.ShapeDtypeStruct((_CT, _E), jnp.float32),
        ],
    )(x, W)


def _pick(ma, ia, mb, ib):
    take_a = (ma > mb) | ((ma == mb) & (ia < ib))
    return jnp.where(take_a, ma, mb), jnp.where(take_a, ia, ib)


def _merge_top2(a, b):
    m1a, i1a, m2a, i2a = a
    m1b, i1b, m2b, i2b = b
    m1, i1 = _pick(m1a, i1a, m1b, i1b)
    a_won = (m1a > m1b) | ((m1a == m1b) & (i1a < i1b))
    c1m = jnp.where(a_won, m2a, m1a)
    c1i = jnp.where(a_won, i2a, i1a)
    c2m = jnp.where(a_won, m1b, m2b)
    c2i = jnp.where(a_won, i1b, i2b)
    m2, i2 = _pick(c1m, c1i, c2m, c2i)
    return m1, i1, m2, i2


def _sc_route(probs2d):
    mesh = plsc.VectorSubcoreMesh(core_axis_name="c", subcore_axis_name="s")

    @functools.partial(
        pl.kernel,
        mesh=mesh,
        compiler_params=pltpu.CompilerParams(needs_layout_passes=False),
        out_type=[
            jax.ShapeDtypeStruct((_CT,), jnp.float32),
            jax.ShapeDtypeStruct((_CT,), jnp.float32),
            jax.ShapeDtypeStruct((_CT,), jnp.int32),
            jax.ShapeDtypeStruct((_CT,), jnp.int32),
        ],
        scratch_types=[
            pltpu.VMEM((_TPW, _E), jnp.float32),
            pltpu.VMEM((_TPW,), jnp.float32),
            pltpu.VMEM((_TPW,), jnp.float32),
            pltpu.VMEM((_TPW,), jnp.int32),
            pltpu.VMEM((_TPW,), jnp.int32),
        ],
    )
    def k(probs_hbm, w1_hbm, w2_hbm, e1_hbm, e2_hbm,
          lv, w1v, w2v, e1v, e2v):
        wid = lax.axis_index("s") * 2 + lax.axis_index("c")
        base = wid * _TPW
        pltpu.sync_copy(probs_hbm.at[pl.ds(base, _TPW)], lv)

        iota = lax.iota(jnp.int32, _L)
        neg = jnp.full((_L,), -1.0, jnp.float32)

        def group_body(g, _):
            rows = g * _L + iota
            chains = []
            for c in range(_NC):
                e0 = c * (_E // _NC)
                m1 = plsc.load_gather(lv, [rows, jnp.full((_L,), e0, jnp.int32)])
                i1 = jnp.full((_L,), e0, jnp.int32)
                m2 = neg
                i2 = i1
                for e in range(e0 + 1, e0 + _E // _NC):
                    v = plsc.load_gather(lv, [rows, jnp.full((_L,), e, jnp.int32)])
                    es = jnp.full((_L,), e, jnp.int32)
                    gt1 = v > m1
                    gt2 = v > m2
                    i2 = jnp.where(gt1, i1, jnp.where(gt2, es, i2))
                    m2 = jnp.where(gt1, m1, jnp.where(gt2, v, m2))
                    i1 = jnp.where(gt1, es, i1)
                    m1 = jnp.where(gt1, v, m1)
                chains.append((m1, i1, m2, i2))
            t01 = _merge_top2(chains[0], chains[1])
            t23 = _merge_top2(chains[2], chains[3])
            m1, i1, m2, i2 = _merge_top2(t01, t23)

            denom = (m1 + m2) + 1e-6
            w1 = m1 / denom
            w2 = m2 / denom

            w1v[pl.ds(g * _L, _L)] = w1
            w2v[pl.ds(g * _L, _L)] = w2
            e1v[pl.ds(g * _L, _L)] = i1
            e2v[pl.ds(g * _L, _L)] = i2
            return 0

        lax.fori_loop(0, _NG, group_body, 0)

        pltpu.sync_copy(w1v, w1_hbm.at[pl.ds(base, _TPW)])
        pltpu.sync_copy(w2v, w2_hbm.at[pl.ds(base, _TPW)])
        pltpu.sync_copy(e1v, e1_hbm.at[pl.ds(base, _TPW)])
        pltpu.sync_copy(e2v, e2_hbm.at[pl.ds(base, _TPW)])

    return k(probs2d)


def kernel(hidden_states, W):
    b, s, h = hidden_states.shape
    x = hidden_states.reshape(b * s, h)
    logits_c, outs_c = [], []
    for c in range(_NCHUNK):
        logits, probs = _tc_logits(x, W, c)
        logits_c.append(logits)
        outs_c.append(_sc_route(probs))
    logits = jnp.concatenate(logits_c, axis=0)
    w1, w2, e1, e2 = (jnp.concatenate([o[i] for o in outs_c]) for i in range(4))
    routing_weights = jnp.stack([w1, w2], axis=-1).reshape(b, s, 2)
    selected_experts = jnp.stack([e1, e2], axis=-1).reshape(b, s, 2)
    router_logits = logits.reshape(b, s, _E)
    return routing_weights, selected_experts, router_logits
